# baseline (speedup 1.0000x reference)
import functools

import jax
import jax.numpy as jnp
from jax import lax
from jax.experimental import pallas as pl
from jax.experimental.pallas import tpu as pltpu
from jax.experimental.pallas import tpu_sc as plsc

VOCAB = 100000
DIM = 128
B = 16384
NC = 2
NS = 16
NW = NC * NS
BPW = B // NW
CHUNK = 128
NCHUNK = BPW // CHUNK
LANE = 16
GROUPS = CHUNK // LANE


def _mesh():
    return plsc.VectorSubcoreMesh(
        core_axis_name="c", subcore_axis_name="s", num_cores=NC, num_subcores=NS
    )


@functools.partial(
    pl.kernel,
    out_type=jax.ShapeDtypeStruct((B,), jnp.float32),
    mesh=_mesh(),
    scratch_types=[
        pltpu.VMEM((BPW,), jnp.int32),
        pltpu.VMEM((BPW,), jnp.int32),
        pltpu.VMEM((BPW,), jnp.float32),
        pltpu.VMEM((BPW,), jnp.float32),
        pltpu.VMEM((64, DIM), jnp.float32),
        pltpu.VMEM((64, DIM), jnp.float32),
        pltpu.VMEM((64, DIM), jnp.float32),
        pltpu.VMEM((64, DIM), jnp.float32),
        pltpu.VMEM((64, DIM), jnp.float32),
        pltpu.VMEM((64, DIM), jnp.float32),
        pltpu.VMEM((64, DIM), jnp.float32),
        pltpu.VMEM((64, DIM), jnp.float32),
        pltpu.VMEM((BPW,), jnp.float32),
        pltpu.SemaphoreType.DMA,
        pltpu.SemaphoreType.DMA,
        pltpu.SemaphoreType.DMA,
        pltpu.SemaphoreType.DMA,
        pltpu.SemaphoreType.DMA,
        pltpu.SemaphoreType.DMA,
    ],
)
def _glove_sc(i_hbm, j_hbm, emb_hbm, ctx_hbm, bias_hbm, cbias_hbm, out_hbm,
              idx_i, idx_j, bi, bj, wi0, wj0, wi1, wj1, wi2, wj2, wi3, wj3,
              outv, sem_w0, sem_w1, sem_w2, sem_w3, sem_bi, sem_bj):
    wid = lax.axis_index("s") * NC + lax.axis_index("c")
    base = wid * BPW

    pltpu.sync_copy(i_hbm.at[pl.ds(base, BPW)], idx_i)
    pltpu.sync_copy(j_hbm.at[pl.ds(base, BPW)], idx_j)

    wbufs = [(wi0, wj0), (wi1, wj1), (wi2, wj2), (wi3, wj3)]
    wsems = [sem_w0, sem_w1, sem_w2, sem_w3]
    NBUF = 4
    SCHED = [(k * 64, 64) for k in range(BPW // 64)]

    def fire_chunk(c):
        wi_b, wj_b = wbufs[c % NBUF]
        sem = wsems[c % NBUF]
        off, size = SCHED[c]
        ii = idx_i.at[pl.ds(off, size)]
        jj = idx_j.at[pl.ds(off, size)]
        ci = pltpu.async_copy(emb_hbm.at[ii], wi_b.at[pl.ds(0, size)], sem)
        cj = pltpu.async_copy(ctx_hbm.at[jj], wj_b.at[pl.ds(0, size)], sem)
        cb_i = pltpu.async_copy(bias_hbm.at[ii], bi.at[pl.ds(off, size)], sem_bi)
        cb_j = pltpu.async_copy(cbias_hbm.at[jj], bj.at[pl.ds(off, size)], sem_bj)
        return ci, cj, cb_i, cb_j

    w_copies = [fire_chunk(0), fire_chunk(1), fire_chunk(2)]

    lane = lax.iota(jnp.int32, LANE)
    perm = {k: lane ^ k for k in (8, 4, 2, 1)}
    mask = {k: (lane & k) == 0 for k in (8, 4, 2, 1)}
    bitrev = [0, 8, 4, 12, 2, 10, 6, 14, 1, 9, 5, 13, 3, 11, 7, 15]

    def shuffle(v, k):
        return v.at[perm[k]].get(mode="promise_in_bounds")

    def merge(a, b, k):
        x = jnp.where(mask[k], a, b)
        y = jnp.where(mask[k], b, a)
        return x + shuffle(y, k)

    for c in range(len(SCHED)):
        wi_b, wj_b = wbufs[c % NBUF]
        ci, cj, cb_i, cb_j = w_copies[c]
        ci.wait()
        cj.wait()
        if c + 3 < len(SCHED):
            w_copies.append(fire_chunk(c + 3))
        cb_i.wait()
        cb_j.wait()
        off, size = SCHED[c]

        def group(g, _, wi_b=wi_b, wj_b=wj_b, off=off):
            rbase = g * LANE

            def kblock(k, accs, wi_b=wi_b, wj_b=wj_b, rbase=rbase):
                col = k * LANE
                return tuple(
                    accs[p]
                    + wi_b[rbase + p, pl.ds(col, LANE)]
                    * wj_b[rbase + p, pl.ds(col, LANE)]
                    for p in range(LANE)
                )

            zero = jnp.zeros((LANE,), jnp.float32)
            accs = lax.fori_loop(0, DIM // LANE, kblock, (zero,) * LANE)
            stack = []
            for p in bitrev:
                node = (16, accs[p])
                while stack and stack[-1][0] == node[0]:
                    lvl, other = stack.pop()
                    node = (lvl // 2, merge(other, node[1], lvl // 2))
                stack.append(node)
            res = stack[0][1]
            ob = off + rbase
            outv[pl.ds(ob, LANE)] = (
                res + bi[pl.ds(ob, LANE)] + bj[pl.ds(ob, LANE)]
            )
            return 0

        lax.fori_loop(0, size // LANE, group, 0)

    pltpu.sync_copy(outv, out_hbm.at[pl.ds(base, BPW)])


def kernel(i, j, embedding, context_embedding, bias, context_bias):
    return _glove_sc(
        jnp.asarray(i, jnp.int32),
        jnp.asarray(j, jnp.int32),
        embedding,
        context_embedding,
        bias,
        context_bias,
    )


# speedup vs baseline: 1.0348x; 1.0348x over previous
import functools

import jax
import jax.numpy as jnp
from jax import lax
from jax.experimental import pallas as pl
from jax.experimental.pallas import tpu as pltpu
from jax.experimental.pallas import tpu_sc as plsc

VOCAB = 100000
DIM = 128
B = 16384
NC = 2
NS = 16
NW = NC * NS
BPW = B // NW
CHUNK = 128
NCHUNK = BPW // CHUNK
LANE = 16
GROUPS = CHUNK // LANE


def _mesh():
    return plsc.VectorSubcoreMesh(
        core_axis_name="c", subcore_axis_name="s", num_cores=NC, num_subcores=NS
    )


@functools.partial(
    pl.kernel,
    out_type=jax.ShapeDtypeStruct((B,), jnp.float32),
    mesh=_mesh(),
    scratch_types=[
        pltpu.VMEM((BPW,), jnp.int32),
        pltpu.VMEM((BPW,), jnp.int32),
        pltpu.VMEM((BPW,), jnp.float32),
        pltpu.VMEM((BPW,), jnp.float32),
        pltpu.VMEM((CHUNK, DIM), jnp.float32),
        pltpu.VMEM((CHUNK, DIM), jnp.float32),
        pltpu.VMEM((CHUNK, DIM), jnp.float32),
        pltpu.VMEM((CHUNK, DIM), jnp.float32),
        pltpu.VMEM((CHUNK, DIM), jnp.float32),
        pltpu.VMEM((CHUNK, DIM), jnp.float32),
        pltpu.VMEM((BPW,), jnp.float32),
        pltpu.SemaphoreType.DMA,
        pltpu.SemaphoreType.DMA,
        pltpu.SemaphoreType.DMA,
        pltpu.SemaphoreType.DMA,
        pltpu.SemaphoreType.DMA,
    ],
)
def _glove_sc(i_hbm, j_hbm, emb_hbm, ctx_hbm, bias_hbm, cbias_hbm, out_hbm,
              idx_i, idx_j, bi, bj, wi0, wj0, wi1, wj1, wi2, wj2, outv,
              sem_w0, sem_w1, sem_w2, sem_bi, sem_bj):
    wid = lax.axis_index("s") * NC + lax.axis_index("c")
    base = wid * BPW

    pltpu.sync_copy(i_hbm.at[pl.ds(base, BPW)], idx_i)
    pltpu.sync_copy(j_hbm.at[pl.ds(base, BPW)], idx_j)

    wbufs = [(wi0, wj0), (wi1, wj1), (wi2, wj2)]
    wsems = [sem_w0, sem_w1, sem_w2]
    NBUF = 3
    SCHED = [(0, 128), (128, 128), (256, 128), (384, 128)]

    def fire_chunk(c):
        wi_b, wj_b = wbufs[c % NBUF]
        sem = wsems[c % NBUF]
        off, size = SCHED[c]
        ii = idx_i.at[pl.ds(off, size)]
        jj = idx_j.at[pl.ds(off, size)]
        ci = pltpu.async_copy(emb_hbm.at[ii], wi_b.at[pl.ds(0, size)], sem)
        cj = pltpu.async_copy(ctx_hbm.at[jj], wj_b.at[pl.ds(0, size)], sem)
        cb_i = pltpu.async_copy(bias_hbm.at[ii], bi.at[pl.ds(off, size)], sem_bi)
        cb_j = pltpu.async_copy(cbias_hbm.at[jj], bj.at[pl.ds(off, size)], sem_bj)
        return ci, cj, cb_i, cb_j

    w_copies = [fire_chunk(0), fire_chunk(1)]

    lane = lax.iota(jnp.int32, LANE)
    perm = {k: lane ^ k for k in (8, 4, 2, 1)}
    mask = {k: (lane & k) == 0 for k in (8, 4, 2, 1)}
    bitrev = [0, 8, 4, 12, 2, 10, 6, 14, 1, 9, 5, 13, 3, 11, 7, 15]

    def shuffle(v, k):
        return v.at[perm[k]].get(mode="promise_in_bounds")

    def merge(a, b, k):
        x = jnp.where(mask[k], a, b)
        y = jnp.where(mask[k], b, a)
        return x + shuffle(y, k)

    for c in range(len(SCHED)):
        wi_b, wj_b = wbufs[c % NBUF]
        ci, cj, cb_i, cb_j = w_copies[c]
        ci.wait()
        cj.wait()
        if c + 2 < len(SCHED):
            w_copies.append(fire_chunk(c + 2))
        cb_i.wait()
        cb_j.wait()
        off, size = SCHED[c]

        def group(g, _, wi_b=wi_b, wj_b=wj_b, off=off):
            rbase = g * LANE

            def kblock(k, accs, wi_b=wi_b, wj_b=wj_b, rbase=rbase):
                col = k * LANE
                return tuple(
                    accs[p]
                    + wi_b[rbase + p, pl.ds(col, LANE)]
                    * wj_b[rbase + p, pl.ds(col, LANE)]
                    for p in range(LANE)
                )

            zero = jnp.zeros((LANE,), jnp.float32)
            accs = lax.fori_loop(0, DIM // LANE - 1, kblock, (zero,) * LANE)
            last = DIM - LANE
            stack = []
            for p in bitrev:
                r = rbase + p
                node = (16, accs[p]
                        + wi_b[r, pl.ds(last, LANE)] * wj_b[r, pl.ds(last, LANE)])
                while stack and stack[-1][0] == node[0]:
                    lvl, other = stack.pop()
                    node = (lvl // 2, merge(other, node[1], lvl // 2))
                stack.append(node)
            res = stack[0][1]
            ob = off + rbase
            outv[pl.ds(ob, LANE)] = (
                res + bi[pl.ds(ob, LANE)] + bj[pl.ds(ob, LANE)]
            )
            return 0

        lax.fori_loop(0, size // LANE, group, 0)

    pltpu.sync_copy(outv, out_hbm.at[pl.ds(base, BPW)])


def kernel(i, j, embedding, context_embedding, bias, context_bias):
    return _glove_sc(
        jnp.asarray(i, jnp.int32),
        jnp.asarray(j, jnp.int32),
        embedding,
        context_embedding,
        bias,
        context_bias,
    )


# speedup vs baseline: 1.0601x; 1.0244x over previous
import functools

import jax
import jax.numpy as jnp
from jax import lax
from jax.experimental import pallas as pl
from jax.experimental.pallas import tpu as pltpu
from jax.experimental.pallas import tpu_sc as plsc

VOCAB = 100000
DIM = 128
B = 16384
NC = 2
NS = 16
NW = NC * NS
BPW = B // NW
CHUNK = 128
NCHUNK = BPW // CHUNK
LANE = 16
GROUPS = CHUNK // LANE


def _mesh():
    return plsc.VectorSubcoreMesh(
        core_axis_name="c", subcore_axis_name="s", num_cores=NC, num_subcores=NS
    )


@functools.partial(
    pl.kernel,
    out_type=jax.ShapeDtypeStruct((B,), jnp.float32),
    mesh=_mesh(),
    scratch_types=[
        pltpu.VMEM((BPW,), jnp.int32),
        pltpu.VMEM((BPW,), jnp.int32),
        pltpu.VMEM((BPW,), jnp.float32),
        pltpu.VMEM((BPW,), jnp.float32),
        pltpu.VMEM((CHUNK, DIM), jnp.float32),
        pltpu.VMEM((CHUNK, DIM), jnp.float32),
        pltpu.VMEM((CHUNK, DIM), jnp.float32),
        pltpu.VMEM((CHUNK, DIM), jnp.float32),
        pltpu.VMEM((CHUNK, DIM), jnp.float32),
        pltpu.VMEM((CHUNK, DIM), jnp.float32),
        pltpu.VMEM((BPW,), jnp.float32),
        pltpu.SemaphoreType.DMA,
        pltpu.SemaphoreType.DMA,
        pltpu.SemaphoreType.DMA,
        pltpu.SemaphoreType.DMA,
        pltpu.SemaphoreType.DMA,
        pltpu.SemaphoreType.DMA,
    ],
)
def _glove_sc(i_hbm, j_hbm, emb_hbm, ctx_hbm, bias_hbm, cbias_hbm, out_hbm,
              idx_i, idx_j, bi, bj, wi0, wj0, wi1, wj1, wi2, wj2, outv,
              sem_w0, sem_w1, sem_w2, sem_bi, sem_bj, sem_out):
    wid = lax.axis_index("s") * NC + lax.axis_index("c")
    base = wid * BPW

    ii_copy = pltpu.async_copy(i_hbm.at[pl.ds(base, BPW)], idx_i, sem_w0)
    jj_copy = pltpu.async_copy(j_hbm.at[pl.ds(base, BPW)], idx_j, sem_w1)
    ii_copy.wait()
    jj_copy.wait()

    wbufs = [(wi0, wj0), (wi1, wj1), (wi2, wj2)]
    wsems = [sem_w0, sem_w1, sem_w2]
    NBUF = 3
    SCHED = [(0, 128), (128, 128), (256, 128), (384, 128)]

    def fire_chunk(c):
        wi_b, wj_b = wbufs[c % NBUF]
        sem = wsems[c % NBUF]
        off, size = SCHED[c]
        ii = idx_i.at[pl.ds(off, size)]
        jj = idx_j.at[pl.ds(off, size)]
        ci = pltpu.async_copy(emb_hbm.at[ii], wi_b.at[pl.ds(0, size)], sem)
        cj = pltpu.async_copy(ctx_hbm.at[jj], wj_b.at[pl.ds(0, size)], sem)
        cb_i = pltpu.async_copy(bias_hbm.at[ii], bi.at[pl.ds(off, size)], sem_bi)
        cb_j = pltpu.async_copy(cbias_hbm.at[jj], bj.at[pl.ds(off, size)], sem_bj)
        return ci, cj, cb_i, cb_j

    w_copies = [fire_chunk(0), fire_chunk(1)]
    out_copies = []

    lane = lax.iota(jnp.int32, LANE)
    perm = {k: lane ^ k for k in (8, 4, 2, 1)}
    mask = {k: (lane & k) == 0 for k in (8, 4, 2, 1)}
    bitrev = [0, 8, 4, 12, 2, 10, 6, 14, 1, 9, 5, 13, 3, 11, 7, 15]

    def shuffle(v, k):
        return v.at[perm[k]].get(mode="promise_in_bounds")

    def merge(a, b, k):
        x = jnp.where(mask[k], a, b)
        y = jnp.where(mask[k], b, a)
        return x + shuffle(y, k)

    for c in range(len(SCHED)):
        wi_b, wj_b = wbufs[c % NBUF]
        ci, cj, cb_i, cb_j = w_copies[c]
        ci.wait()
        cj.wait()
        if c + 2 < len(SCHED):
            w_copies.append(fire_chunk(c + 2))
        cb_i.wait()
        cb_j.wait()
        off, size = SCHED[c]

        def group(g, _, wi_b=wi_b, wj_b=wj_b, off=off):
            rbase = g * LANE

            def kblock(k, accs, wi_b=wi_b, wj_b=wj_b, rbase=rbase):
                col = k * LANE
                return tuple(
                    accs[p]
                    + wi_b[rbase + p, pl.ds(col, LANE)]
                    * wj_b[rbase + p, pl.ds(col, LANE)]
                    for p in range(LANE)
                )

            zero = jnp.zeros((LANE,), jnp.float32)
            accs = lax.fori_loop(0, DIM // LANE, kblock, (zero,) * LANE)
            stack = []
            for p in bitrev:
                node = (16, accs[p])
                while stack and stack[-1][0] == node[0]:
                    lvl, other = stack.pop()
                    node = (lvl // 2, merge(other, node[1], lvl // 2))
                stack.append(node)
            res = stack[0][1]
            ob = off + rbase
            outv[pl.ds(ob, LANE)] = (
                res + bi[pl.ds(ob, LANE)] + bj[pl.ds(ob, LANE)]
            )
            return 0

        lax.fori_loop(0, size // LANE, group, 0)
        out_copies.append(
            pltpu.async_copy(
                outv.at[pl.ds(off, size)],
                out_hbm.at[pl.ds(base + off, size)],
                sem_out,
            )
        )

    for oc in out_copies:
        oc.wait()


def kernel(i, j, embedding, context_embedding, bias, context_bias):
    return _glove_sc(
        jnp.asarray(i, jnp.int32),
        jnp.asarray(j, jnp.int32),
        embedding,
        context_embedding,
        bias,
        context_bias,
    )
